# baseline (speedup 1.0000x reference)
import jax
import jax.numpy as jnp
import numpy as np
from jax import lax
from jax.experimental import pallas as pl
from jax.experimental.pallas import tpu as pltpu
from jax.experimental.pallas import tpu_sc as plsc

N = 10000
E = 320000
D = 128
DA = D + 16
MAXD = 4

NC = 2
NS = 16
NW = NC * NS
CH = 128
CPT = 80
EPT = CPT * CH
EPAD = NW * EPT
RPT = 640
NPAD = NS * RPT
DUMMY = NPAD - 1

GSPLIT = 4

BLK = 512


def _sc_body(x_hbm, row_hbm, col_hbm, sums_hbm, cnt_hbm,
             shared_acc, row_v, col_v, buf_a, buf_b,
             gsem0, gsem1, gsem2, gsem3, gsem4, gsem5, gsem6, gsem7,
             ssem_a, ssem_b):
    c = lax.axis_index("c")
    s = lax.axis_index("s")
    wid = c * NS + s
    zero16 = jnp.zeros((16,), jnp.float32)
    ones16 = jnp.ones((16,), jnp.float32)
    NG = CPT // 8

    def fill(ref, val16):
        def body(i, carry):
            for k in range(D // 16):
                ref[i, pl.ds(k * 16, 16)] = val16
            return carry
        lax.fori_loop(0, CH, body, 0)

    def zero_my_slice(src):
        for k in range(RPT // CH):
            pltpu.sync_copy(src, shared_acc.at[pl.ds(s * RPT + k * CH, CH)])

    def read_my_slice(out_hbm, stage):
        for k in range(RPT // CH):
            off = s * RPT + k * CH
            pltpu.sync_copy(shared_acc.at[pl.ds(off, CH)], stage)
            pltpu.sync_copy(stage, out_hbm.at[c, pl.ds(off, CH)])

    def load_idx(g, slot, rows=True):
        dst = pl.ds(slot * 8, 8)
        src = pl.ds(wid * CPT + g * 8, 8)
        if rows:
            pltpu.sync_copy(row_hbm.at[src], row_v.at[dst])
        pltpu.sync_copy(col_hbm.at[src], col_v.at[dst])

    bufs = (buf_a, buf_b)
    gsems = ((gsem0, gsem1, gsem2, gsem3), (gsem4, gsem5, gsem6, gsem7))
    ssems = (ssem_a, ssem_b)

    GH = CH // GSPLIT

    def gather(g, slot, j):
        for h in range(GSPLIT):
            pltpu.async_copy(
                x_hbm.at[row_v.at[slot * 8 + j, pl.ds(h * GH, GH)]],
                bufs[j % 2].at[pl.ds(h * GH, GH)], gsems[j % 2][h])

    def gwait(j):
        for h in range(GSPLIT):
            pltpu.make_async_copy(x_hbm.at[row_v.at[0, pl.ds(0, GH)]],
                                  bufs[j % 2].at[pl.ds(0, GH)],
                                  gsems[j % 2][h]).wait()

    def scatter(slot, j):
        pltpu.async_copy(bufs[j % 2],
                         shared_acc.at[col_v.at[slot * 8 + j]],
                         ssems[j % 2], add=True)

    def sdrain(sem, n=1):
        for _ in range(n):
            pltpu.make_async_copy(buf_a, shared_acc.at[pl.ds(0, CH)],
                                  sem).wait()

    fill(buf_a, zero16)
    zero_my_slice(buf_a)
    plsc.subcore_barrier()


    load_idx(0, 0)
    gather(0, 0, 0)
    for j in range(8):
        if j >= 1:
            sdrain(ssems[(j + 1) % 2])
        if j < 7:
            gather(0, 0, j + 1)
        else:
            load_idx(1, 1)
            gather(1, 1, 0)
        gwait(j)
        scatter(0, j)

    def group_a(g, carry):
        slot = lax.rem(g, 2)
        for j in range(8):
            sdrain(ssems[(j + 1) % 2])
            if j < 7:
                gather(g, slot, j + 1)
            else:
                @pl.when(g < NG - 1)
                def _():
                    load_idx(g + 1, 1 - slot)
                    gather(g + 1, 1 - slot, 0)
            gwait(j)
            scatter(slot, j)
        return carry
    lax.fori_loop(1, NG, group_a, 0)
    sdrain(ssems[7 % 2])

    plsc.subcore_barrier()
    read_my_slice(sums_hbm, buf_a)

    fill(buf_a, zero16)
    zero_my_slice(buf_a)
    fill(buf_a, ones16)
    plsc.subcore_barrier()

    def fire8(slot):
        for j in range(8):
            pltpu.async_copy(buf_a, shared_acc.at[col_v.at[slot * 8 + j]],
                             gsem0, add=True)

    load_idx(0, 0, rows=False)
    fire8(0)

    def group_b(g, carry):
        slot = lax.rem(g, 2)

        @pl.when(g >= 2)
        def _():
            sdrain(gsem0, 8)
        load_idx(g, slot, rows=False)
        fire8(slot)
        return carry
    lax.fori_loop(1, NG, group_b, 0)
    sdrain(gsem0, 16)

    plsc.subcore_barrier()
    read_my_slice(cnt_hbm, buf_b)


def _segment_sum_sc(x, row2d, col2d, interpret=False):
    mesh = plsc.VectorSubcoreMesh(core_axis_name="c", subcore_axis_name="s")
    return pl.kernel(
        _sc_body,
        out_type=[
            jax.ShapeDtypeStruct((NC, NPAD, D), jnp.float32),
            jax.ShapeDtypeStruct((NC, NPAD, D), jnp.float32),
        ],
        mesh=mesh,
        scratch_types=[
            pltpu.VMEM_SHARED((NPAD, D), jnp.float32),
            pltpu.VMEM((16, CH), jnp.int32),
            pltpu.VMEM((16, CH), jnp.int32),
            pltpu.VMEM((CH, D), jnp.float32),
            pltpu.VMEM((CH, D), jnp.float32),
        ] + [pltpu.SemaphoreType.DMA] * 10,
        interpret=interpret,
    )(x, row2d, col2d)


def _tc_body(x_ref, sums_ref, cnt_ref, w_ref, b_ref, out_ref):
    sums = sums_ref[0] + sums_ref[1]
    cnt = cnt_ref[0, :, 0:1] + cnt_ref[1, :, 0:1]
    neigh = jnp.where(cnt > 0, sums / jnp.maximum(cnt, 1.0), 0.0)
    deg = jnp.minimum(cnt, np.float32(MAXD))
    h = jnp.concatenate([neigh, x_ref[...]], axis=1)
    acc = jnp.zeros((BLK, D), jnp.float32)
    for b in range(MAXD + 1):
        vals = jnp.dot(h, w_ref[b], preferred_element_type=jnp.float32)
        vals = vals + b_ref[b][None, :]
        acc = acc + jnp.where(deg == np.float32(b), vals, 0.0)
    out_ref[...] = acc


def _bucket_mm(x_pad, sums, cnt, w_cat, bias, interpret=False):
    return pl.pallas_call(
        _tc_body,
        grid=(NPAD // BLK,),
        in_specs=[
            pl.BlockSpec((BLK, D), lambda i: (i, 0)),
            pl.BlockSpec((NC, BLK, D), lambda i: (0, i, 0)),
            pl.BlockSpec((NC, BLK, D), lambda i: (0, i, 0)),
            pl.BlockSpec((MAXD + 1, 2 * D, D), lambda i: (0, 0, 0)),
            pl.BlockSpec((MAXD + 1, D), lambda i: (0, 0)),
        ],
        out_specs=pl.BlockSpec((BLK, D), lambda i: (i, 0)),
        out_shape=jax.ShapeDtypeStruct((NPAD, D), jnp.float32),
        interpret=interpret,
    )(x_pad, sums, cnt, w_cat, bias)


@jax.jit
def kernel(x, edge_index, neigh_W, neigh_b, root_W, root_b):
    row, col = edge_index[0], edge_index[1]
    pad = EPAD - E
    row_p = jnp.concatenate(
        [row, jnp.zeros((pad,), jnp.int32)]).reshape(EPAD // CH, CH)
    col_p = jnp.concatenate(
        [col, jnp.full((pad,), DUMMY, jnp.int32)]).reshape(EPAD // CH, CH)
    sums, cnt = _segment_sum_sc(x, row_p, col_p)
    x_pad = jnp.concatenate([x, jnp.zeros((NPAD - N, D), x.dtype)])
    w_cat = jnp.concatenate([neigh_W, root_W], axis=1)
    bias = neigh_b + root_b
    out = _bucket_mm(x_pad, sums, cnt, w_cat, bias)
    return out[:N]


# speedup vs baseline: 1.0066x; 1.0066x over previous
import jax
import jax.numpy as jnp
import numpy as np
from jax import lax
from jax.experimental import pallas as pl
from jax.experimental.pallas import tpu as pltpu
from jax.experimental.pallas import tpu_sc as plsc

N = 10000
E = 320000
D = 128
DA = D + 16
MAXD = 4

NC = 2
NS = 16
NW = NC * NS
CH = 128
CPT = 80
EPT = CPT * CH
EPAD = NW * EPT
RPT = 640
NPAD = NS * RPT
DUMMY = NPAD - 1

GSPLIT = 4

BLK = 512


def _sc_body(x_hbm, row_hbm, col_hbm, sums_hbm, cnt_hbm,
             shared_acc, row_v, col_v, buf_a, buf_b,
             gsem0, gsem1, gsem2, gsem3, gsem4, gsem5, gsem6, gsem7,
             ssem_a, ssem_b):
    c = lax.axis_index("c")
    s = lax.axis_index("s")
    wid = c * NS + s
    zero16 = jnp.zeros((16,), jnp.float32)
    ones16 = jnp.ones((16,), jnp.float32)
    NG = CPT // 8

    def fill(ref, val16):
        def body(i, carry):
            for k in range(D // 16):
                ref[i, pl.ds(k * 16, 16)] = val16
            return carry
        lax.fori_loop(0, CH, body, 0)

    def zero_my_slice(src):
        for k in range(RPT // CH):
            pltpu.sync_copy(src, shared_acc.at[pl.ds(s * RPT + k * CH, CH)])

    def read_my_slice(out_hbm, stage):
        for k in range(RPT // CH):
            off = s * RPT + k * CH
            pltpu.sync_copy(shared_acc.at[pl.ds(off, CH)], stage)
            pltpu.sync_copy(stage, out_hbm.at[c, pl.ds(off, CH)])

    def load_idx(g, slot, rows=True):
        dst = pl.ds(slot * 8, 8)
        src = pl.ds(wid * CPT + g * 8, 8)
        if rows:
            pltpu.sync_copy(row_hbm.at[src], row_v.at[dst])
        pltpu.sync_copy(col_hbm.at[src], col_v.at[dst])

    bufs = (buf_a, buf_b)
    gsems = ((gsem0, gsem1, gsem2, gsem3), (gsem4, gsem5, gsem6, gsem7))
    ssems = (ssem_a, ssem_b)

    GH = CH // GSPLIT

    def gather(g, slot, j):
        for h in range(GSPLIT):
            pltpu.async_copy(
                x_hbm.at[row_v.at[slot * 8 + j, pl.ds(h * GH, GH)]],
                bufs[j % 2].at[pl.ds(h * GH, GH)], gsems[j % 2][h])

    def gwait(j):
        for h in range(GSPLIT):
            pltpu.make_async_copy(x_hbm.at[row_v.at[0, pl.ds(0, GH)]],
                                  bufs[j % 2].at[pl.ds(0, GH)],
                                  gsems[j % 2][h]).wait()

    def scatter(slot, j):
        pltpu.async_copy(bufs[j % 2],
                         shared_acc.at[col_v.at[slot * 8 + j]],
                         ssems[j % 2], add=True)

    def sdrain(sem, n=1):
        for _ in range(n):
            pltpu.make_async_copy(buf_a, shared_acc.at[pl.ds(0, CH)],
                                  sem).wait()

    fill(buf_a, zero16)
    zero_my_slice(buf_a)
    plsc.subcore_barrier()


    load_idx(0, 0)
    gather(0, 0, 0)
    for j in range(8):
        if j >= 1:
            sdrain(ssems[(j + 1) % 2])
        if j < 7:
            gather(0, 0, j + 1)
        else:
            load_idx(1, 1)
            gather(1, 1, 0)
        gwait(j)
        scatter(0, j)

    def group_a(g, carry):
        slot = lax.rem(g, 2)
        for j in range(8):
            sdrain(ssems[(j + 1) % 2])
            if j < 7:
                gather(g, slot, j + 1)
            else:
                @pl.when(g < NG - 1)
                def _():
                    load_idx(g + 1, 1 - slot)
                    gather(g + 1, 1 - slot, 0)
            gwait(j)
            scatter(slot, j)
        return carry
    lax.fori_loop(1, NG, group_a, 0)
    sdrain(ssems[7 % 2])

    plsc.subcore_barrier()
    read_my_slice(sums_hbm, buf_a)

    fill(buf_a, zero16)
    zero_my_slice(buf_a)
    fill(buf_a, ones16)
    plsc.subcore_barrier()

    def fire8(slot):
        for j in range(8):
            pltpu.async_copy(buf_a, shared_acc.at[col_v.at[slot * 8 + j]],
                             gsem0, add=True)

    load_idx(0, 0, rows=False)
    fire8(0)

    def group_b(g, carry):
        slot = lax.rem(g, 2)

        @pl.when(g >= 2)
        def _():
            sdrain(gsem0, 8)
        load_idx(g, slot, rows=False)
        fire8(slot)
        return carry
    lax.fori_loop(1, NG, group_b, 0)
    sdrain(gsem0, 16)

    plsc.subcore_barrier()
    read_my_slice(cnt_hbm, buf_b)


def _segment_sum_sc(x, row2d, col2d, interpret=False):
    mesh = plsc.VectorSubcoreMesh(core_axis_name="c", subcore_axis_name="s")
    return pl.kernel(
        _sc_body,
        out_type=[
            jax.ShapeDtypeStruct((NC, NPAD, D), jnp.float32),
            jax.ShapeDtypeStruct((NC, NPAD, D), jnp.float32),
        ],
        mesh=mesh,
        scratch_types=[
            pltpu.VMEM_SHARED((NPAD, D), jnp.float32),
            pltpu.VMEM((16, CH), jnp.int32),
            pltpu.VMEM((16, CH), jnp.int32),
            pltpu.VMEM((CH, D), jnp.float32),
            pltpu.VMEM((CH, D), jnp.float32),
        ] + [pltpu.SemaphoreType.DMA] * 10,
        interpret=interpret,
    )(x, row2d, col2d)


def _tc_body(x_ref, sums_ref, cnt_ref, w_ref, b_ref, out_ref):
    sums = sums_ref[0] + sums_ref[1]
    cnt = cnt_ref[0, :, 0:1] + cnt_ref[1, :, 0:1]
    neigh = jnp.where(cnt > 0, sums / jnp.maximum(cnt, 1.0), 0.0)
    deg = jnp.minimum(cnt, np.float32(MAXD))
    h = jnp.concatenate([neigh, x_ref[...]], axis=1)
    acc = jnp.zeros((BLK, D), jnp.float32)
    for b in range(MAXD + 1):
        vals = jnp.dot(h, w_ref[b], preferred_element_type=jnp.float32)
        vals = vals + b_ref[b][None, :]
        acc = acc + jnp.where(deg == np.float32(b), vals, 0.0)
    out_ref[...] = acc


def _bucket_mm(x, sums, cnt, w_cat, bias, interpret=False):
    return pl.pallas_call(
        _tc_body,
        grid=(NPAD // BLK,),
        in_specs=[
            pl.BlockSpec((BLK, D), lambda i: (i, 0)),
            pl.BlockSpec((NC, BLK, D), lambda i: (0, i, 0)),
            pl.BlockSpec((NC, BLK, D), lambda i: (0, i, 0)),
            pl.BlockSpec((MAXD + 1, 2 * D, D), lambda i: (0, 0, 0)),
            pl.BlockSpec((MAXD + 1, D), lambda i: (0, 0)),
        ],
        out_specs=pl.BlockSpec((BLK, D), lambda i: (i, 0)),
        out_shape=jax.ShapeDtypeStruct((N, D), jnp.float32),
        interpret=interpret,
    )(x, sums, cnt, w_cat, bias)


@jax.jit
def kernel(x, edge_index, neigh_W, neigh_b, root_W, root_b):
    row, col = edge_index[0], edge_index[1]
    pad = EPAD - E
    row_p = jnp.concatenate(
        [row, jnp.zeros((pad,), jnp.int32)]).reshape(EPAD // CH, CH)
    col_p = jnp.concatenate(
        [col, jnp.full((pad,), DUMMY, jnp.int32)]).reshape(EPAD // CH, CH)
    sums, cnt = _segment_sum_sc(x, row_p, col_p)
    w_cat = jnp.concatenate([neigh_W, root_W], axis=1)
    bias = neigh_b + root_b
    return _bucket_mm(x, sums, cnt, w_cat, bias)
